# Initial kernel scaffold; baseline (speedup 1.0000x reference)
import jax
import jax.numpy as jnp
from jax.experimental import pallas as pl


def kernel(sensor_ids, weight):
    raise NotImplementedError("write your pallas kernel here")


# speedup vs baseline: 6.5109x; 6.5109x over previous
import functools

import jax
import jax.numpy as jnp
from jax import lax
from jax.experimental import pallas as pl
from jax.experimental.pallas import tpu as pltpu
from jax.experimental.pallas import tpu_sc as plsc

EMBED_D = 128
NUM_WORKERS = 32
GATHER_ROWS = 128


def _make_sc_gather(num_rows: int):
    rows_per_w = num_rows // NUM_WORKERS
    chunks = rows_per_w // GATHER_ROWS
    assert chunks % 2 == 0

    mesh = plsc.VectorSubcoreMesh(core_axis_name="c", subcore_axis_name="s")

    @functools.partial(
        pl.kernel,
        mesh=mesh,
        out_type=jax.ShapeDtypeStruct((num_rows, EMBED_D), jnp.float32),
        scratch_types=[
            pltpu.VMEM((chunks, GATHER_ROWS), jnp.int32),
            pltpu.VMEM((GATHER_ROWS, EMBED_D), jnp.float32),
            pltpu.VMEM((GATHER_ROWS, EMBED_D), jnp.float32),
            pltpu.SemaphoreType.DMA,
            pltpu.SemaphoreType.DMA,
        ],
    )
    def k(ids_hbm, w_hbm, out_hbm, idx_v, rows0, rows1, sem0, sem1):
        wid = lax.axis_index("s") * 2 + lax.axis_index("c")
        base = wid * rows_per_w
        pltpu.sync_copy(ids_hbm.at[wid], idx_v)

        pltpu.async_copy(w_hbm.at[idx_v.at[0]], rows0, sem0)

        def body(i, carry):
            j = i * 2
            pltpu.make_async_copy(w_hbm.at[idx_v.at[j]], rows0, sem0).wait()
            pltpu.async_copy(w_hbm.at[idx_v.at[j + 1]], rows1, sem1)
            pltpu.sync_copy(rows0, out_hbm.at[pl.ds(base + j * GATHER_ROWS,
                                                    GATHER_ROWS)])

            @pl.when(j + 2 < chunks)
            def _():
                pltpu.async_copy(w_hbm.at[idx_v.at[j + 2]], rows0, sem0)

            pltpu.make_async_copy(w_hbm.at[idx_v.at[j + 1]], rows1, sem1).wait()
            pltpu.sync_copy(rows1, out_hbm.at[pl.ds(base + (j + 1) * GATHER_ROWS,
                                                    GATHER_ROWS)])
            return carry

        lax.fori_loop(0, chunks // 2, body, 0, unroll=False)

    return k


def kernel(sensor_ids, weight):
    b, t = sensor_ids.shape
    num_rows = b * t
    ids = sensor_ids.astype(jnp.int32).reshape(
        NUM_WORKERS, num_rows // (NUM_WORKERS * GATHER_ROWS), GATHER_ROWS)
    out = _make_sc_gather(num_rows)(ids, weight)
    return out.reshape(b, t, EMBED_D)


# speedup vs baseline: 15.5430x; 2.3872x over previous
import functools

import jax
import jax.numpy as jnp
from jax import lax
from jax.experimental import pallas as pl
from jax.experimental.pallas import tpu as pltpu
from jax.experimental.pallas import tpu_sc as plsc

EMBED_D = 128
NUM_WORKERS = 32
GATHER_ROWS = 128
TABLE_PAD = 1024


def _make_sc_gather(num_rows: int):
    rows_per_w = num_rows // NUM_WORKERS
    chunks = rows_per_w // GATHER_ROWS
    assert chunks % 2 == 0
    slab = TABLE_PAD // 16

    mesh = plsc.VectorSubcoreMesh(core_axis_name="c", subcore_axis_name="s")

    @functools.partial(
        pl.kernel,
        mesh=mesh,
        out_type=jax.ShapeDtypeStruct((num_rows, EMBED_D), jnp.float32),
        scratch_types=[
            pltpu.VMEM_SHARED((TABLE_PAD, EMBED_D), jnp.float32),
            pltpu.VMEM((chunks, GATHER_ROWS), jnp.int32),
            pltpu.VMEM((GATHER_ROWS, EMBED_D), jnp.float32),
            pltpu.VMEM((GATHER_ROWS, EMBED_D), jnp.float32),
            pltpu.SemaphoreType.DMA,
            pltpu.SemaphoreType.DMA,
        ],
    )
    def k(ids_hbm, w_hbm, out_hbm, table_sh, idx_v, rows0, rows1, sem0, sem1):
        cid = lax.axis_index("c")
        sid = lax.axis_index("s")
        wid = sid * 2 + cid
        base = wid * rows_per_w

        pltpu.sync_copy(w_hbm.at[pl.ds(sid * slab, slab)],
                        table_sh.at[pl.ds(sid * slab, slab)])
        pltpu.sync_copy(ids_hbm.at[wid], idx_v)
        plsc.subcore_barrier()

        pltpu.async_copy(table_sh.at[idx_v.at[0]], rows0, sem0)

        def body(i, carry):
            j = i * 2
            pltpu.make_async_copy(table_sh.at[idx_v.at[j]], rows0, sem0).wait()
            pltpu.async_copy(table_sh.at[idx_v.at[j + 1]], rows1, sem1)
            pltpu.sync_copy(rows0, out_hbm.at[pl.ds(base + j * GATHER_ROWS,
                                                    GATHER_ROWS)])

            @pl.when(j + 2 < chunks)
            def _():
                pltpu.async_copy(table_sh.at[idx_v.at[j + 2]], rows0, sem0)

            pltpu.make_async_copy(table_sh.at[idx_v.at[j + 1]], rows1, sem1).wait()
            pltpu.sync_copy(rows1, out_hbm.at[pl.ds(base + (j + 1) * GATHER_ROWS,
                                                    GATHER_ROWS)])
            return carry

        lax.fori_loop(0, chunks // 2, body, 0, unroll=False)

    return k


def kernel(sensor_ids, weight):
    b, t = sensor_ids.shape
    num_rows = b * t
    ids = sensor_ids.astype(jnp.int32).reshape(
        NUM_WORKERS, num_rows // (NUM_WORKERS * GATHER_ROWS), GATHER_ROWS)
    w_pad = jnp.pad(weight, ((0, TABLE_PAD - weight.shape[0]), (0, 0)))
    out = _make_sc_gather(num_rows)(ids, w_pad)
    return out.reshape(b, t, EMBED_D)
